# baseline (speedup 1.0000x reference)
import jax
import jax.numpy as jnp
from jax.experimental import pallas as pl
from jax.experimental.pallas import tpu as pltpu

_K = 5
_H = 512
_W = 512
_NM = 2


def _bf(v):
    return v.astype(jnp.bfloat16).astype(jnp.float32)


def _kmeans_body(mask_ref, scores_ref, out_ref):
    yi = jax.lax.broadcasted_iota(jnp.int32, (_H, _W), 0)
    xi = jax.lax.broadcasted_iota(jnp.int32, (_H, _W), 1)
    y = yi.astype(jnp.float32)
    x = xi.astype(jnp.float32)
    pidx = yi * _W + xi
    big = jnp.int32(2 ** 30)
    c2 = y * y + x * x
    yb = _bf(y)
    xb = _bf(x)

    ms = [mask_ref[j] for j in range(_NM)]
    fgs = [m != 0.0 for m in ms]
    fgfs = [jnp.where(fg, 1.0, 0.0) for fg in fgs]
    counts = [jnp.sum(fgf) for fgf in fgfs]

    init_cy = [[] for _ in range(_NM)]
    init_cx = [[] for _ in range(_NM)]
    ss = [scores_ref[j] + jnp.where(fgs[j], 10.0, 0.0) for j in range(_NM)]
    for _ in range(_K):
        for j in range(_NM):
            mx = jnp.max(ss[j])
            p0 = jnp.min(jnp.where(ss[j] == mx, pidx, big))
            init_cy[j].append((p0 // _W).astype(jnp.float32))
            init_cx[j].append((p0 % _W).astype(jnp.float32))
            ss[j] = jnp.where(pidx == p0, -jnp.inf, ss[j])

    def distances(cys, cxs):
        ds = []
        for k in range(_K):
            cy, cx = cys[k], cxs[k]
            cent2 = cy * cy + cx * cx
            dot = yb * _bf(cy) + xb * _bf(cx)
            d2 = (c2 + cent2) - 2.0 * dot
            ds.append(jnp.maximum(d2, 0.0))
        return ds

    def argmin5(ds):
        best = ds[0]
        bk = jnp.zeros((_H, _W), jnp.int32)
        for k in range(1, _K):
            lt = ds[k] < best
            best = jnp.where(lt, ds[k], best)
            bk = jnp.where(lt, k, bk)
        return best, bk

    ones_row = jnp.ones((8, _H), jnp.float32)
    ones_col = jnp.ones((_W, 8), jnp.float32)
    xv = jax.lax.broadcasted_iota(jnp.int32, (8, _W), 1).astype(jnp.float32)
    yv = jax.lax.broadcasted_iota(jnp.int32, (_H, 8), 0).astype(jnp.float32)
    xvb = _bf(xv[0:1, :])
    yvb = _bf(yv[:, 0:1])

    def update_one(fg, fgf, cys, cxs):
        ds = distances(cys, cxs)
        best, bk = argmin5(ds)
        cnts = []
        sxs = []
        sys_ = []
        for k in range(_K):
            self_k = jnp.where(bk == k, fgf, 0.0)
            colcnt = jnp.dot(ones_row, self_k)
            rowcnt = jnp.dot(self_k, ones_col)
            cnts.append(jnp.sum(colcnt[0:1, :]))
            sxs.append(jnp.sum(colcnt[0:1, :] * xvb))
            sys_.append(jnp.sum(rowcnt[:, 0:1] * yvb))

        any_empty = jnp.minimum(
            jnp.minimum(jnp.minimum(cnts[0], cnts[1]), jnp.minimum(cnts[2], cnts[3])),
            cnts[4],
        ) <= 0.0

        def _far(_):
            mind = jnp.where(fg, best, -jnp.inf)
            mm = jnp.max(mind)
            pf = jnp.min(jnp.where(mind == mm, pidx, big))
            return (pf // _W).astype(jnp.float32), (pf % _W).astype(jnp.float32)

        fy, fx = jax.lax.cond(any_empty, _far, lambda _: (0.0, 0.0), None)

        ncy = []
        ncx = []
        for k in range(_K):
            cnt = cnts[k]
            denom = jnp.maximum(cnt, 1.0)
            nonempty = cnt > 0.0
            ncy.append(jnp.where(nonempty, sys_[k] / denom, fy))
            ncx.append(jnp.where(nonempty, sxs[k] / denom, fx))
        return tuple(ncy), tuple(ncx)

    def update(it, carry):
        return tuple(
            update_one(fgs[j], fgfs[j], carry[j][0], carry[j][1])
            for j in range(_NM)
        )

    carry0 = tuple((tuple(init_cy[j]), tuple(init_cx[j])) for j in range(_NM))
    carry = jax.lax.fori_loop(0, 9, update, carry0)

    for j in range(_NM):
        cys, cxs = carry[j]
        _, bk = argmin5(distances(cys, cxs))
        special = counts[j] <= float(_K)
        for k in range(_K):
            vk = jnp.where(fgs[j] & (bk == k), 1.0, 0.0)
            if k == 0:
                out_ref[j, k] = jnp.where(special, ms[j], vk)
            else:
                out_ref[j, k] = jnp.where(special, 0.0, vk)


@jax.jit
def kernel(binary_mask):
    x = binary_mask
    if x.ndim == 5 and x.shape[1] == 1:
        x = x[:, 0]
    B, C, H, W = x.shape
    n = B * C
    masks = x.reshape(n, H, W)
    keys = jax.random.split(jax.random.key(42), n)
    scores = jax.vmap(lambda k: jax.random.uniform(k, (H * W,)))(keys)
    scores = scores.reshape(n, H, W)

    steps = n // _NM
    out = pl.pallas_call(
        _kmeans_body,
        grid=(steps,),
        in_specs=[
            pl.BlockSpec((_NM, H, W), lambda i: (i, 0, 0)),
            pl.BlockSpec((_NM, H, W), lambda i: (i, 0, 0)),
        ],
        out_specs=pl.BlockSpec((_NM, _K, H, W), lambda i: (i, 0, 0, 0)),
        out_shape=jax.ShapeDtypeStruct((n, _K, H, W), masks.dtype),
        compiler_params=pltpu.CompilerParams(
            dimension_semantics=("parallel",),
        ),
    )(masks, scores)
    return out.reshape(B, C, _K, H, W)
